# baseline (speedup 1.0000x reference)
import functools

import jax
import jax.numpy as jnp
from jax.experimental import pallas as pl

_B = 4096
_H = 128
_HEADS = 4
_SCALE = (_H // _HEADS) ** -0.5
_TILE = 1024


def _fused_kernel(title_ref, content_ref, W1_ref, b1_ref, W2_ref, b2_ref,
                  tok_ref, pos_ref, Wqkv_ref, Wout_ref, bout_ref,
                  Wcls_ref, bcls_ref, out_ref):
    f32 = jnp.float32

    def hsum(v):
        r = v.shape[0]
        return jnp.sum(v.reshape(r, _HEADS, _H), axis=2)

    pos0 = pos_ref[0:1, :]
    pos1 = pos_ref[1:2, :]
    t0 = tok_ref[...] + pos0
    qkv0 = jax.lax.dot(t0, Wqkv_ref[...], preferred_element_type=f32)
    q0 = qkv0[:, 0:512]
    k0 = qkv0[:, 512:1024]
    v0 = qkv0[:, 1024:1536]
    Wk = Wqkv_ref[:, 512:1024]
    Wv = Wqkv_ref[:, 1024:1536]

    d0 = hsum(q0 * k0) * _SCALE
    U = hsum(Wk * q0) * _SCALE

    PT = jax.lax.dot_general(
        Wcls_ref[...], Wout_ref[...],
        dimension_numbers=(((0,), (1,)), ((), ())),
        preferred_element_type=f32)
    G0 = hsum(Wv * PT[0:1, :])
    G1 = hsum(Wv * PT[1:2, :])
    c00 = hsum(v0 * PT[0:1, :])
    c01 = hsum(v0 * PT[1:2, :])

    dc = jax.lax.dot(pos1, U, preferred_element_type=f32) - d0
    g0c = jax.lax.dot(pos1, G0, preferred_element_type=f32) - c00
    g1c = jax.lax.dot(pos1, G1, preferred_element_type=f32) - c01
    bz = (jax.lax.dot(bout_ref[...], Wcls_ref[...], preferred_element_type=f32)
          + bcls_ref[...])
    base0 = bz[:, 0:1] + jnp.sum(c00, axis=1, keepdims=True)
    base1 = bz[:, 1:2] + jnp.sum(c01, axis=1, keepdims=True)

    a = jax.lax.dot(title_ref[...], W1_ref[...], preferred_element_type=f32) + b1_ref[...]
    b = jax.lax.dot(content_ref[...], W2_ref[...], preferred_element_type=f32) + b2_ref[...]
    x = jnp.maximum(jnp.concatenate([a, b], axis=1), 0.0)

    F = jnp.concatenate([U, G0, G1], axis=1)
    y = jax.lax.dot(x, F, preferred_element_type=f32)
    w = jax.nn.sigmoid(y[:, 0:4] + dc)
    g0 = y[:, 4:8] + g0c
    g1 = y[:, 8:12] + g1c
    o0 = base0 + jnp.sum(w * g0, axis=1, keepdims=True)
    o1 = base1 + jnp.sum(w * g1, axis=1, keepdims=True)
    out_ref[...] = jnp.concatenate([o0, o1], axis=1)


@functools.partial(jax.jit, static_argnames=())
def _run(news_title, news_content, W_nt_t, b_nt_t, W_nt_c, b_nt_c,
         token, pos_emb, W_qkv, W_out, b_out, W_cls, b_cls):
    T = _TILE
    grid = (_B // T,)
    row_spec = pl.BlockSpec((T, 768), lambda i: (i, 0))

    def rep(shape):
        return pl.BlockSpec(shape, lambda i: tuple(0 for _ in shape))

    return pl.pallas_call(
        _fused_kernel,
        grid=grid,
        in_specs=[
            row_spec, row_spec,
            rep((768, 64)), rep((1, 64)),
            rep((768, 64)), rep((1, 64)),
            rep((1, 128)), rep((2, 128)),
            rep((128, 1536)),
            rep((512, 128)), rep((1, 128)),
            rep((128, 2)), rep((1, 2)),
        ],
        out_specs=pl.BlockSpec((T, 2), lambda i: (i, 0)),
        out_shape=jax.ShapeDtypeStruct((_B, 2), jnp.float32),
    )(news_title, news_content,
      W_nt_t, b_nt_t.reshape(1, 64), W_nt_c, b_nt_c.reshape(1, 64),
      token.reshape(1, 128), pos_emb.reshape(2, 128), W_qkv,
      W_out, b_out.reshape(1, 128), W_cls, b_cls.reshape(1, 2))


def kernel(news_title, news_content, tweet_content, tweet_profile, user_profile, user_description, source_description, W_nt_t, b_nt_t, W_nt_c, b_nt_c, W_tw_c, b_tw_c, W_tw_p, b_tw_p, W_us_p, b_us_p, W_us_d, b_us_d, W_sr_d, b_sr_d, W_rte_nt, b_rte_nt, W_rte_tu, b_rte_tu, W_rte_uu, b_rte_uu, W_rte_ns, b_rte_ns, token, pos_emb, W_qkv, W_out, b_out, W_cls, b_cls, t_news, t_tweet, t_user, t_source, ei_nt, ei_tu, ei_uu, ei_ns):
    return _run(news_title, news_content, W_nt_t, b_nt_t, W_nt_c, b_nt_c,
                token, pos_emb, W_qkv, W_out, b_out, W_cls, b_cls)
